# Initial kernel scaffold; baseline (speedup 1.0000x reference)
import jax
import jax.numpy as jnp
from jax.experimental import pallas as pl


def kernel(node_id_orig, node_id_dest, edge_index, edge_label_index, emb, c1_to_Wl, c1_to_bl, c1_to_Wr, c1_rev_Wl, c1_rev_bl, c1_rev_Wr, c2_to_Wl, c2_to_bl, c2_to_Wr, c2_rev_Wl, c2_rev_bl, c2_rev_Wr):
    raise NotImplementedError("write your pallas kernel here")


# speedup vs baseline: 3.6717x; 3.6717x over previous
import functools

import jax
import jax.numpy as jnp
from jax import lax
from jax.experimental import pallas as pl
from jax.experimental.pallas import tpu as pltpu
from jax.experimental.pallas import tpu_sc as plsc

_N = 10000
_E = 320000
_EL = 320000
_H = 128
_V = 128
_NC = 2
_NS = 16
_L = 16

_EPT = _E // _NS
_CH = 80
_NCH = _EPT // _CH
_RPT = _N // _NS
_RB = 640
_RBL = _N - (_NS - 1) * _RB
_ELW = _EL // (_NC * _NS)
_CCH = 80
_NCC = _ELW // _CCH
_ABLK = 25
_CHB = 40
_BBLK = 50
_NBB = _EPT // (_BBLK * _CHB)

_mesh = plsc.VectorSubcoreMesh(core_axis_name="c", subcore_axis_name="s")
_f32 = jnp.float32
_i32 = jnp.int32


@functools.partial(
    pl.kernel,
    out_type=jax.ShapeDtypeStruct((_NC * _N * _V,), _f32),
    mesh=_mesh,
    compiler_params=pltpu.CompilerParams(needs_layout_passes=False),
    scratch_types=[
        pltpu.VMEM((_N,), _i32),
        pltpu.VMEM((_ABLK * _CH,), _i32),
        pltpu.VMEM((_ABLK * _CH,), _i32),
        pltpu.VMEM((_CH,), _i32),
        pltpu.VMEM((_CH,), _f32),
        pltpu.VMEM_SHARED((_N * _V,), _f32),
    ],
)
def _sc_counts(eidx_hbm, sidx_hbm, nidcat_hbm, zeros_hbm, ccat_hbm,
               nid_v, key_v, row_v, flat_v, ones_v, acc):
    c = lax.axis_index("c")
    s = lax.axis_index("s")
    base = c * _E + s * _EPT
    pltpu.sync_copy(zeros_hbm.at[pl.ds(s * _RPT * _V, _RPT * _V)],
                    acc.at[pl.ds(s * _RPT * _V, _RPT * _V)])
    pltpu.sync_copy(nidcat_hbm.at[pl.ds(c * _N, _N)], nid_v)
    for i in range(_CH // _L):
        ones_v[pl.ds(i * _L, _L)] = jnp.ones((_L,), _f32)
    plsc.subcore_barrier()

    def block(b, carry):
        bbase = base + b * _ABLK * _CH
        pltpu.sync_copy(eidx_hbm.at[pl.ds(bbase, _ABLK * _CH)], key_v)
        pltpu.sync_copy(sidx_hbm.at[pl.ds(bbase, _ABLK * _CH)], row_v)

        def chunk(j, carry2):
            for i in range(_CH // _L):
                off = j * _CH + i * _L
                ev = key_v[pl.ds(off, _L)]
                rv = row_v[pl.ds(off, _L)]
                vv = plsc.load_gather(nid_v, [ev])
                flat_v[pl.ds(i * _L, _L)] = rv * _V + vv
            pltpu.sync_copy(ones_v, acc.at[flat_v], add=True)
            return carry2

        lax.fori_loop(0, _ABLK, chunk, 0)
        return carry

    lax.fori_loop(0, _NCH // _ABLK, block, 0)
    plsc.subcore_barrier()
    pltpu.sync_copy(acc.at[pl.ds(s * _RPT * _V, _RPT * _V)],
                    ccat_hbm.at[pl.ds(c * _N * _V + s * _RPT * _V, _RPT * _V)])


@functools.partial(
    pl.kernel,
    out_type=jax.ShapeDtypeStruct((_NC * _N, _H), _f32),
    mesh=_mesh,
    compiler_params=pltpu.CompilerParams(needs_layout_passes=False),
    scratch_types=[
        pltpu.VMEM((_BBLK, _CHB), _i32),
        pltpu.VMEM((_BBLK, _CHB), _i32),
        pltpu.VMEM((_CHB, _H), _f32),
        pltpu.VMEM((_CHB, _H), _f32),
        pltpu.SemaphoreType.DMA,
        pltpu.SemaphoreType.DMA,
        pltpu.VMEM_SHARED((_N, _H), _f32),
    ],
)
def _sc_segsum(hcat_hbm, gidx_hbm, sidx_hbm, zeros2_hbm, scat_hbm,
               gidx_v, sidx_v, rows0, rows1, sem0, sem1, acc):
    c = lax.axis_index("c")
    s = lax.axis_index("s")

    @pl.when(s < _NS - 1)
    def _():
        pltpu.sync_copy(zeros2_hbm.at[pl.ds(s * _RB, _RB)],
                        acc.at[pl.ds(s * _RB, _RB)])

    @pl.when(s == _NS - 1)
    def _():
        pltpu.sync_copy(zeros2_hbm.at[pl.ds((_NS - 1) * _RB, _RBL)],
                        acc.at[pl.ds((_NS - 1) * _RB, _RBL)])
    plsc.subcore_barrier()

    dummy = hcat_hbm.at[pl.ds(0, _CHB)]

    def block(b, carry):
        pltpu.sync_copy(gidx_hbm.at[c, s, b], gidx_v)
        pltpu.sync_copy(sidx_hbm.at[c, s, b], sidx_v)
        pltpu.async_copy(hcat_hbm.at[gidx_v.at[0]], rows0, sem0)

        def pair(jj, carry2):
            j0 = jj * 2
            pltpu.make_async_copy(dummy, rows0, sem0).wait()
            pltpu.async_copy(hcat_hbm.at[gidx_v.at[j0 + 1]], rows1, sem1)
            pltpu.sync_copy(rows0, acc.at[sidx_v.at[j0]], add=True)
            pltpu.make_async_copy(dummy, rows1, sem1).wait()

            @pl.when(jj < _BBLK // 2 - 1)
            def _():
                pltpu.async_copy(hcat_hbm.at[gidx_v.at[j0 + 2]], rows0, sem0)

            pltpu.sync_copy(rows1, acc.at[sidx_v.at[j0 + 1]], add=True)
            return carry2

        lax.fori_loop(0, _BBLK // 2, pair, 0)
        return carry

    lax.fori_loop(0, _NBB, block, 0)
    plsc.subcore_barrier()

    @pl.when(s < _NS - 1)
    def _():
        pltpu.sync_copy(acc.at[pl.ds(s * _RB, _RB)],
                        scat_hbm.at[pl.ds(c * _N + s * _RB, _RB)])

    @pl.when(s == _NS - 1)
    def _():
        pltpu.sync_copy(acc.at[pl.ds((_NS - 1) * _RB, _RBL)],
                        scat_hbm.at[pl.ds(c * _N + (_NS - 1) * _RB, _RBL)])


@functools.partial(
    pl.kernel,
    out_type=jax.ShapeDtypeStruct((_EL,), _f32),
    mesh=_mesh,
    compiler_params=pltpu.CompilerParams(needs_layout_passes=False),
    scratch_types=[
        pltpu.VMEM((_NCC, _CCH), _i32),
        pltpu.VMEM((_NCC, _CCH), _i32),
        pltpu.VMEM((_CCH, _H), _f32),
        pltpu.VMEM((_CCH, _H), _f32),
        pltpu.VMEM((_CCH, _H), _f32),
        pltpu.VMEM((_CCH, _H), _f32),
        pltpu.VMEM((_ELW,), _f32),
        pltpu.SemaphoreType.DMA,
        pltpu.SemaphoreType.DMA,
    ],
)
def _sc_dot(gcat_hbm, aidx_hbm, bidx_hbm, out_hbm,
            aidx_v, bidx_v, a0, b0, a1, b1, out_v, sem_a, sem_b):
    c = lax.axis_index("c")
    s = lax.axis_index("s")
    pltpu.sync_copy(aidx_hbm.at[c, s], aidx_v)
    pltpu.sync_copy(bidx_hbm.at[c, s], bidx_v)
    dummy = gcat_hbm.at[pl.ds(0, _CCH)]
    pltpu.async_copy(gcat_hbm.at[aidx_v.at[0]], a0, sem_a)
    pltpu.async_copy(gcat_hbm.at[bidx_v.at[0]], b0, sem_b)

    lane = lax.iota(_i32, _L)

    def compute(abuf, bbuf, j):
        for blk in range(_CCH // _L):
            res = jnp.zeros((_L,), _f32)
            for t in range(_L):
                e = blk * _L + t
                acc = abuf[e, pl.ds(0, _L)] * bbuf[e, pl.ds(0, _L)]
                for k in range(1, _H // _L):
                    acc = acc + abuf[e, pl.ds(k * _L, _L)] * bbuf[e, pl.ds(k * _L, _L)]
                res = jnp.where(lane == t, jnp.sum(acc), res)
            out_v[pl.ds(j * _CCH + blk * _L, _L)] = res

    def pair(jj, carry):
        j0 = jj * 2
        pltpu.make_async_copy(dummy, a0, sem_a).wait()
        pltpu.make_async_copy(dummy, b0, sem_b).wait()
        pltpu.async_copy(gcat_hbm.at[aidx_v.at[j0 + 1]], a1, sem_a)
        pltpu.async_copy(gcat_hbm.at[bidx_v.at[j0 + 1]], b1, sem_b)
        compute(a0, b0, j0)
        pltpu.make_async_copy(dummy, a1, sem_a).wait()
        pltpu.make_async_copy(dummy, b1, sem_b).wait()

        @pl.when(jj < _NCC // 2 - 1)
        def _():
            pltpu.async_copy(gcat_hbm.at[aidx_v.at[j0 + 2]], a0, sem_a)
            pltpu.async_copy(gcat_hbm.at[bidx_v.at[j0 + 2]], b0, sem_b)

        compute(a1, b1, j0 + 1)
        return carry

    lax.fori_loop(0, _NCC // 2, pair, 0)
    jt = _NCC - 1
    pltpu.async_copy(gcat_hbm.at[aidx_v.at[jt]], a0, sem_a)
    pltpu.async_copy(gcat_hbm.at[bidx_v.at[jt]], b0, sem_b)
    pltpu.make_async_copy(dummy, a0, sem_a).wait()
    pltpu.make_async_copy(dummy, b0, sem_b).wait()
    compute(a0, b0, jt)
    pltpu.sync_copy(out_v, out_hbm.at[pl.ds((c * _NS + s) * _ELW, _ELW)])


_NB = 10
_BR = _N // _NB


def _tc1_body(c_ref, nid_ref, emb_ref, wl_ref, bl_ref, wr_ref, h_ref):
    emb = emb_ref[...]
    m = jnp.dot(emb, wl_ref[0].T, precision="highest")
    r = jnp.dot(emb, wr_ref[0].T, precision="highest")
    cnt = c_ref[0]
    deg = jnp.maximum(jnp.sum(cnt, axis=1, keepdims=True), 1.0)
    nid = nid_ref[0, 0, 0]
    oh = (nid[:, None] == lax.broadcasted_iota(_i32, (1, _V), 1)).astype(_f32)
    h = (jnp.dot(cnt, m, precision="highest") / deg + bl_ref[0]
         + jnp.dot(oh, r, precision="highest"))
    h_ref[0] = jnp.maximum(h, 0.0)


_tc1 = pl.pallas_call(
    _tc1_body,
    grid=(2, _NB),
    in_specs=[
        pl.BlockSpec((1, _BR, _V), lambda d, i: (d, i, 0)),
        pl.BlockSpec((1, 1, 1, _BR), lambda d, i: (d, i, 0, 0)),
        pl.BlockSpec((_V, _H), lambda d, i: (0, 0)),
        pl.BlockSpec((1, _H, _H), lambda d, i: (d, 0, 0)),
        pl.BlockSpec((1, 1, _H), lambda d, i: (d, 0, 0)),
        pl.BlockSpec((1, _H, _H), lambda d, i: (d, 0, 0)),
    ],
    out_specs=pl.BlockSpec((1, _BR, _H), lambda d, i: (d, i, 0)),
    out_shape=jax.ShapeDtypeStruct((2, _N, _H), _f32),
)


def _tc2_body(s_ref, h_ref, c_ref, wl_ref, bl_ref, wr_ref, g_ref):
    cnt = c_ref[0]
    deg = jnp.maximum(jnp.sum(cnt, axis=1, keepdims=True), 1.0)
    g = (jnp.dot(s_ref[0], wl_ref[0].T, precision="highest") / deg + bl_ref[0]
         + jnp.dot(h_ref[0], wr_ref[0].T, precision="highest"))
    g_ref[0] = g


_tc2 = pl.pallas_call(
    _tc2_body,
    grid=(2, _NB),
    in_specs=[
        pl.BlockSpec((1, _BR, _H), lambda d, i: (d, i, 0)),
        pl.BlockSpec((1, _BR, _H), lambda d, i: (d, i, 0)),
        pl.BlockSpec((1, _BR, _V), lambda d, i: (d, i, 0)),
        pl.BlockSpec((1, _H, _H), lambda d, i: (d, 0, 0)),
        pl.BlockSpec((1, 1, _H), lambda d, i: (d, 0, 0)),
        pl.BlockSpec((1, _H, _H), lambda d, i: (d, 0, 0)),
    ],
    out_specs=pl.BlockSpec((1, _BR, _H), lambda d, i: (d, i, 0)),
    out_shape=jax.ShapeDtypeStruct((2, _N, _H), _f32),
)


def kernel(node_id_orig, node_id_dest, edge_index, edge_label_index, emb,
           c1_to_Wl, c1_to_bl, c1_to_Wr,
           c1_rev_Wl, c1_rev_bl, c1_rev_Wr,
           c2_to_Wl, c2_to_bl, c2_to_Wr,
           c2_rev_Wl, c2_rev_bl, c2_rev_Wr):
    edge_index = edge_index.astype(_i32)
    edge_label_index = edge_label_index.astype(_i32)
    src, dst = edge_index[0], edge_index[1]

    eidx_flat = edge_index.reshape(2 * _E)
    sidx_flat = edge_index[::-1].reshape(2 * _E)
    nidcat = jnp.concatenate([node_id_orig, node_id_dest]).astype(_i32)
    zeros_flat = jnp.zeros((_N * _V,), _f32)
    ccat = _sc_counts(eidx_flat, sidx_flat, nidcat, zeros_flat)
    c_cat = ccat.reshape(2, _N, _V)

    nid_stack = jnp.stack([node_id_dest, node_id_orig]).astype(_i32)
    nid_stack = nid_stack.reshape(2, _NB, 1, _BR)
    wl1 = jnp.stack([c1_to_Wl, c1_rev_Wl])
    bl1 = jnp.stack([c1_to_bl, c1_rev_bl]).reshape(2, 1, _H)
    wr1 = jnp.stack([c1_to_Wr, c1_rev_Wr])
    h_cat = _tc1(c_cat, nid_stack, emb, wl1, bl1, wr1)

    gidx = jnp.concatenate([src + _N, dst]).reshape(_NC, _NS, _NBB, _BBLK, _CHB)
    sidx3 = sidx_flat.reshape(_NC, _NS, _NBB, _BBLK, _CHB)
    zeros2 = jnp.zeros((_N, _H), _f32)
    s_cat = _sc_segsum(h_cat.reshape(2 * _N, _H), gidx, sidx3, zeros2)

    wl2 = jnp.stack([c2_to_Wl, c2_rev_Wl])
    bl2 = jnp.stack([c2_to_bl, c2_rev_bl]).reshape(2, 1, _H)
    wr2 = jnp.stack([c2_to_Wr, c2_rev_Wr])
    g_cat = _tc2(s_cat.reshape(2, _N, _H), h_cat, c_cat, wl2, bl2, wr2)

    aidx = (edge_label_index[0] + _N).reshape(_NC, _NS, _NCC, _CCH)
    bidx = edge_label_index[1].reshape(_NC, _NS, _NCC, _CCH)
    out = _sc_dot(g_cat.reshape(2 * _N, _H), aidx, bidx)
    return out


# speedup vs baseline: 3.9459x; 1.0747x over previous
import functools

import jax
import jax.numpy as jnp
from jax import lax
from jax.experimental import pallas as pl
from jax.experimental.pallas import tpu as pltpu
from jax.experimental.pallas import tpu_sc as plsc

_N = 10000
_E = 320000
_EL = 320000
_H = 128
_V = 128
_NC = 2
_NS = 16
_L = 16

_EPT = _E // _NS
_CH = 80
_NCH = _EPT // _CH
_RPT = _N // _NS
_RB = 640
_RBL = _N - (_NS - 1) * _RB
_ELW = _EL // (_NC * _NS)
_CCH = 80
_NCC = _ELW // _CCH
_ABLK = 25
_CHB = 80
_BBLK = 10
_NBB = _EPT // (_BBLK * _CHB)

_mesh = plsc.VectorSubcoreMesh(core_axis_name="c", subcore_axis_name="s")
_f32 = jnp.float32
_i32 = jnp.int32


@functools.partial(
    pl.kernel,
    out_type=jax.ShapeDtypeStruct((_NC * _N * _V,), _f32),
    mesh=_mesh,
    compiler_params=pltpu.CompilerParams(needs_layout_passes=False),
    scratch_types=[
        pltpu.VMEM((_N,), _i32),
        pltpu.VMEM((_ABLK * _CH,), _i32),
        pltpu.VMEM((_ABLK * _CH,), _i32),
        pltpu.VMEM((_CH,), _i32),
        pltpu.VMEM((_CH,), _f32),
        pltpu.VMEM_SHARED((_N * _V,), _f32),
    ],
)
def _sc_counts(eidx_hbm, sidx_hbm, nidcat_hbm, zeros_hbm, ccat_hbm,
               nid_v, key_v, row_v, flat_v, ones_v, acc):
    c = lax.axis_index("c")
    s = lax.axis_index("s")
    base = c * _E + s * _EPT
    pltpu.sync_copy(zeros_hbm.at[pl.ds(s * _RPT * _V, _RPT * _V)],
                    acc.at[pl.ds(s * _RPT * _V, _RPT * _V)])
    pltpu.sync_copy(nidcat_hbm.at[pl.ds(c * _N, _N)], nid_v)
    for i in range(_CH // _L):
        ones_v[pl.ds(i * _L, _L)] = jnp.ones((_L,), _f32)
    plsc.subcore_barrier()

    def block(b, carry):
        bbase = base + b * _ABLK * _CH
        pltpu.sync_copy(eidx_hbm.at[pl.ds(bbase, _ABLK * _CH)], key_v)
        pltpu.sync_copy(sidx_hbm.at[pl.ds(bbase, _ABLK * _CH)], row_v)

        def chunk(j, carry2):
            for i in range(_CH // _L):
                off = j * _CH + i * _L
                ev = key_v[pl.ds(off, _L)]
                rv = row_v[pl.ds(off, _L)]
                vv = plsc.load_gather(nid_v, [ev])
                flat_v[pl.ds(i * _L, _L)] = rv * _V + vv
            pltpu.sync_copy(ones_v, acc.at[flat_v], add=True)
            return carry2

        lax.fori_loop(0, _ABLK, chunk, 0)
        return carry

    lax.fori_loop(0, _NCH // _ABLK, block, 0)
    plsc.subcore_barrier()
    pltpu.sync_copy(acc.at[pl.ds(s * _RPT * _V, _RPT * _V)],
                    ccat_hbm.at[pl.ds(c * _N * _V + s * _RPT * _V, _RPT * _V)])


@functools.partial(
    pl.kernel,
    out_type=jax.ShapeDtypeStruct((_NC * _N, _H), _f32),
    mesh=_mesh,
    compiler_params=pltpu.CompilerParams(needs_layout_passes=False),
    scratch_types=[
        pltpu.VMEM((_BBLK, _CHB), _i32),
        pltpu.VMEM((_BBLK, _CHB), _i32),
        pltpu.VMEM((_CHB, _H), _f32),
        pltpu.VMEM((_CHB, _H), _f32),
        pltpu.SemaphoreType.DMA,
        pltpu.SemaphoreType.DMA,
        pltpu.VMEM_SHARED((_N, _H), _f32),
    ],
)
def _sc_segsum(hcat_hbm, gidx_hbm, sidx_hbm, zeros2_hbm, scat_hbm,
               gidx_v, sidx_v, rows0, rows1, sem0, sem1, acc):
    c = lax.axis_index("c")
    s = lax.axis_index("s")

    @pl.when(s < _NS - 1)
    def _():
        pltpu.sync_copy(zeros2_hbm.at[pl.ds(s * _RB, _RB)],
                        acc.at[pl.ds(s * _RB, _RB)])

    @pl.when(s == _NS - 1)
    def _():
        pltpu.sync_copy(zeros2_hbm.at[pl.ds((_NS - 1) * _RB, _RBL)],
                        acc.at[pl.ds((_NS - 1) * _RB, _RBL)])
    plsc.subcore_barrier()

    dummy = hcat_hbm.at[pl.ds(0, _CHB)]

    def block(b, carry):
        pltpu.sync_copy(gidx_hbm.at[c, s, b], gidx_v)
        pltpu.sync_copy(sidx_hbm.at[c, s, b], sidx_v)
        pltpu.async_copy(hcat_hbm.at[gidx_v.at[0]], rows0, sem0)

        def pair(jj, carry2):
            j0 = jj * 2
            pltpu.make_async_copy(dummy, rows0, sem0).wait()
            pltpu.async_copy(hcat_hbm.at[gidx_v.at[j0 + 1]], rows1, sem1)
            pltpu.sync_copy(rows0, acc.at[sidx_v.at[j0]], add=True)
            pltpu.make_async_copy(dummy, rows1, sem1).wait()

            @pl.when(jj < _BBLK // 2 - 1)
            def _():
                pltpu.async_copy(hcat_hbm.at[gidx_v.at[j0 + 2]], rows0, sem0)

            pltpu.sync_copy(rows1, acc.at[sidx_v.at[j0 + 1]], add=True)
            return carry2

        lax.fori_loop(0, _BBLK // 2, pair, 0)
        return carry

    lax.fori_loop(0, _NBB, block, 0)
    plsc.subcore_barrier()

    @pl.when(s < _NS - 1)
    def _():
        pltpu.sync_copy(acc.at[pl.ds(s * _RB, _RB)],
                        scat_hbm.at[pl.ds(c * _N + s * _RB, _RB)])

    @pl.when(s == _NS - 1)
    def _():
        pltpu.sync_copy(acc.at[pl.ds((_NS - 1) * _RB, _RBL)],
                        scat_hbm.at[pl.ds(c * _N + (_NS - 1) * _RB, _RBL)])


@functools.partial(
    pl.kernel,
    out_type=jax.ShapeDtypeStruct((_EL,), _f32),
    mesh=_mesh,
    compiler_params=pltpu.CompilerParams(needs_layout_passes=False),
    scratch_types=[
        pltpu.VMEM((_NCC, _CCH), _i32),
        pltpu.VMEM((_NCC, _CCH), _i32),
        pltpu.VMEM((_CCH, _H), _f32),
        pltpu.VMEM((_CCH, _H), _f32),
        pltpu.VMEM((_CCH, _H), _f32),
        pltpu.VMEM((_CCH, _H), _f32),
        pltpu.VMEM((_ELW,), _f32),
        pltpu.SemaphoreType.DMA,
        pltpu.SemaphoreType.DMA,
    ],
)
def _sc_dot(gcat_hbm, aidx_hbm, bidx_hbm, out_hbm,
            aidx_v, bidx_v, a0, b0, a1, b1, out_v, sem_a, sem_b):
    c = lax.axis_index("c")
    s = lax.axis_index("s")
    pltpu.sync_copy(aidx_hbm.at[c, s], aidx_v)
    pltpu.sync_copy(bidx_hbm.at[c, s], bidx_v)
    dummy = gcat_hbm.at[pl.ds(0, _CCH)]
    pltpu.async_copy(gcat_hbm.at[aidx_v.at[0]], a0, sem_a)
    pltpu.async_copy(gcat_hbm.at[bidx_v.at[0]], b0, sem_b)

    lane = lax.iota(_i32, _L)

    def compute(abuf, bbuf, j):
        for blk in range(_CCH // _L):
            res = jnp.zeros((_L,), _f32)
            for t in range(_L):
                e = blk * _L + t
                acc = abuf[e, pl.ds(0, _L)] * bbuf[e, pl.ds(0, _L)]
                for k in range(1, _H // _L):
                    acc = acc + abuf[e, pl.ds(k * _L, _L)] * bbuf[e, pl.ds(k * _L, _L)]
                res = jnp.where(lane == t, jnp.sum(acc), res)
            out_v[pl.ds(j * _CCH + blk * _L, _L)] = res

    def pair(jj, carry):
        j0 = jj * 2
        pltpu.make_async_copy(dummy, a0, sem_a).wait()
        pltpu.make_async_copy(dummy, b0, sem_b).wait()
        pltpu.async_copy(gcat_hbm.at[aidx_v.at[j0 + 1]], a1, sem_a)
        pltpu.async_copy(gcat_hbm.at[bidx_v.at[j0 + 1]], b1, sem_b)
        compute(a0, b0, j0)
        pltpu.make_async_copy(dummy, a1, sem_a).wait()
        pltpu.make_async_copy(dummy, b1, sem_b).wait()

        @pl.when(jj < _NCC // 2 - 1)
        def _():
            pltpu.async_copy(gcat_hbm.at[aidx_v.at[j0 + 2]], a0, sem_a)
            pltpu.async_copy(gcat_hbm.at[bidx_v.at[j0 + 2]], b0, sem_b)

        compute(a1, b1, j0 + 1)
        return carry

    lax.fori_loop(0, _NCC // 2, pair, 0)
    jt = _NCC - 1
    pltpu.async_copy(gcat_hbm.at[aidx_v.at[jt]], a0, sem_a)
    pltpu.async_copy(gcat_hbm.at[bidx_v.at[jt]], b0, sem_b)
    pltpu.make_async_copy(dummy, a0, sem_a).wait()
    pltpu.make_async_copy(dummy, b0, sem_b).wait()
    compute(a0, b0, jt)
    pltpu.sync_copy(out_v, out_hbm.at[pl.ds((c * _NS + s) * _ELW, _ELW)])


_NB = 10
_BR = _N // _NB


def _tc1_body(c_ref, nid_ref, emb_ref, wl_ref, bl_ref, wr_ref, h_ref):
    emb = emb_ref[...]
    m = jnp.dot(emb, wl_ref[0].T, precision="highest")
    r = jnp.dot(emb, wr_ref[0].T, precision="highest")
    cnt = c_ref[0]
    deg = jnp.maximum(jnp.sum(cnt, axis=1, keepdims=True), 1.0)
    nid = nid_ref[0, 0, 0]
    oh = (nid[:, None] == lax.broadcasted_iota(_i32, (1, _V), 1)).astype(_f32)
    h = (jnp.dot(cnt, m, precision="highest") / deg + bl_ref[0]
         + jnp.dot(oh, r, precision="highest"))
    h_ref[0] = jnp.maximum(h, 0.0)


_tc1 = pl.pallas_call(
    _tc1_body,
    grid=(2, _NB),
    in_specs=[
        pl.BlockSpec((1, _BR, _V), lambda d, i: (d, i, 0)),
        pl.BlockSpec((1, 1, 1, _BR), lambda d, i: (d, i, 0, 0)),
        pl.BlockSpec((_V, _H), lambda d, i: (0, 0)),
        pl.BlockSpec((1, _H, _H), lambda d, i: (d, 0, 0)),
        pl.BlockSpec((1, 1, _H), lambda d, i: (d, 0, 0)),
        pl.BlockSpec((1, _H, _H), lambda d, i: (d, 0, 0)),
    ],
    out_specs=pl.BlockSpec((1, _BR, _H), lambda d, i: (d, i, 0)),
    out_shape=jax.ShapeDtypeStruct((2, _N, _H), _f32),
)


def _tc2_body(s_ref, h_ref, c_ref, wl_ref, bl_ref, wr_ref, g_ref):
    cnt = c_ref[0]
    deg = jnp.maximum(jnp.sum(cnt, axis=1, keepdims=True), 1.0)
    g = (jnp.dot(s_ref[0], wl_ref[0].T, precision="highest") / deg + bl_ref[0]
         + jnp.dot(h_ref[0], wr_ref[0].T, precision="highest"))
    g_ref[0] = g


_tc2 = pl.pallas_call(
    _tc2_body,
    grid=(2, _NB),
    in_specs=[
        pl.BlockSpec((1, _BR, _H), lambda d, i: (d, i, 0)),
        pl.BlockSpec((1, _BR, _H), lambda d, i: (d, i, 0)),
        pl.BlockSpec((1, _BR, _V), lambda d, i: (d, i, 0)),
        pl.BlockSpec((1, _H, _H), lambda d, i: (d, 0, 0)),
        pl.BlockSpec((1, 1, _H), lambda d, i: (d, 0, 0)),
        pl.BlockSpec((1, _H, _H), lambda d, i: (d, 0, 0)),
    ],
    out_specs=pl.BlockSpec((1, _BR, _H), lambda d, i: (d, i, 0)),
    out_shape=jax.ShapeDtypeStruct((2, _N, _H), _f32),
)


def kernel(node_id_orig, node_id_dest, edge_index, edge_label_index, emb,
           c1_to_Wl, c1_to_bl, c1_to_Wr,
           c1_rev_Wl, c1_rev_bl, c1_rev_Wr,
           c2_to_Wl, c2_to_bl, c2_to_Wr,
           c2_rev_Wl, c2_rev_bl, c2_rev_Wr):
    edge_index = edge_index.astype(_i32)
    edge_label_index = edge_label_index.astype(_i32)
    src, dst = edge_index[0], edge_index[1]

    eidx_flat = edge_index.reshape(2 * _E)
    sidx_flat = edge_index[::-1].reshape(2 * _E)
    nidcat = jnp.concatenate([node_id_orig, node_id_dest]).astype(_i32)
    zeros_flat = jnp.zeros((_N * _V,), _f32)
    ccat = _sc_counts(eidx_flat, sidx_flat, nidcat, zeros_flat)
    c_cat = ccat.reshape(2, _N, _V)

    nid_stack = jnp.stack([node_id_dest, node_id_orig]).astype(_i32)
    nid_stack = nid_stack.reshape(2, _NB, 1, _BR)
    wl1 = jnp.stack([c1_to_Wl, c1_rev_Wl])
    bl1 = jnp.stack([c1_to_bl, c1_rev_bl]).reshape(2, 1, _H)
    wr1 = jnp.stack([c1_to_Wr, c1_rev_Wr])
    h_cat = _tc1(c_cat, nid_stack, emb, wl1, bl1, wr1)

    gidx = jnp.concatenate([src + _N, dst]).reshape(_NC, _NS, _NBB, _BBLK, _CHB)
    sidx3 = sidx_flat.reshape(_NC, _NS, _NBB, _BBLK, _CHB)
    zeros2 = jnp.zeros((_N, _H), _f32)
    s_cat = _sc_segsum(h_cat.reshape(2 * _N, _H), gidx, sidx3, zeros2)

    wl2 = jnp.stack([c2_to_Wl, c2_rev_Wl])
    bl2 = jnp.stack([c2_to_bl, c2_rev_bl]).reshape(2, 1, _H)
    wr2 = jnp.stack([c2_to_Wr, c2_rev_Wr])
    g_cat = _tc2(s_cat.reshape(2, _N, _H), h_cat, c_cat, wl2, bl2, wr2)

    aidx = (edge_label_index[0] + _N).reshape(_NC, _NS, _NCC, _CCH)
    bidx = edge_label_index[1].reshape(_NC, _NS, _NCC, _CCH)
    out = _sc_dot(g_cat.reshape(2 * _N, _H), aidx, bidx)
    return out
